# baseline (speedup 1.0000x reference)
import functools

import jax
import jax.numpy as jnp
from jax import lax
from jax.experimental import pallas as pl
from jax.experimental.pallas import tpu as pltpu
from jax.experimental.pallas import tpu_sc as plsc

_NC, _NS = 2, 16
_NW = _NC * _NS
_CHUNK = 128
_BLK = 2048


def _make_sc_gather(n_pad: int, emb: int, n_chunks: int):
    b_per_w = n_chunks * _CHUNK
    mesh = plsc.VectorSubcoreMesh(
        core_axis_name="c", subcore_axis_name="s",
        num_cores=_NC, num_subcores=_NS,
    )

    @functools.partial(
        pl.kernel,
        mesh=mesh,
        compiler_params=pltpu.CompilerParams(use_tc_tiling_on_sc=False),
        out_type=jax.ShapeDtypeStruct((n_pad, emb), jnp.float32),
        scratch_types=[
            pltpu.VMEM((b_per_w,), jnp.int32),
            pltpu.VMEM((b_per_w, emb), jnp.float32),
            pltpu.SemaphoreType.DMA,
        ],
    )
    def gather(idx_hbm, table_hbm, out_hbm, idx_v, rows_v, sem):
        wid = lax.axis_index("s") * _NC + lax.axis_index("c")
        pltpu.sync_copy(idx_hbm.at[pl.ds(wid * b_per_w, b_per_w)], idx_v)

        def body(j, carry):
            pltpu.async_copy(
                table_hbm.at[idx_v.at[pl.ds(j * _CHUNK, _CHUNK)]],
                rows_v.at[pl.ds(j * _CHUNK, _CHUNK)],
                sem,
            ).wait()
            return carry

        lax.fori_loop(0, n_chunks, body, 0)
        pltpu.sync_copy(rows_v, out_hbm.at[pl.ds(wid * b_per_w, b_per_w)])

    return gather


def _tc_body(e_ref, f_ref, d_ref, c_ref, we_ref, wf_ref, wd_ref, wc_ref,
             b_ref, g_ref, bet_ref, out_ref):
    h = jnp.dot(e_ref[...], we_ref[...], preferred_element_type=jnp.float32)
    h = h + jnp.dot(f_ref[...], wf_ref[...], preferred_element_type=jnp.float32)
    h = h + jnp.dot(d_ref[...], wd_ref[...], preferred_element_type=jnp.float32)
    h = h + jnp.dot(c_ref[...], wc_ref[...], preferred_element_type=jnp.float32)
    h = h + b_ref[...]
    h = h * (1.0 / (1.0 + jnp.exp(-h)))
    mu = jnp.mean(h, axis=-1, keepdims=True)
    hc = h - mu
    var = jnp.mean(hc * hc, axis=-1, keepdims=True)
    out_ref[...] = hc * lax.rsqrt(var + 1e-5) * g_ref[...] + bet_ref[...]


def _sc_gather(oc, table, n_pad, n_chunks):
    return _make_sc_gather(n_pad, table.shape[1], n_chunks)(oc, table)


def kernel(op_code, features, configs, dim_features, table, W, b, gamma, beta):
    n = features.shape[0]
    ne = table.shape[1]
    nf = features.shape[1]
    nd = dim_features.shape[1]
    ncf = configs.shape[1]
    out_ch = W.shape[1]

    n_chunks = -(-n // (_NW * _CHUNK))
    n_pad = _NW * _CHUNK * n_chunks
    oc = op_code.reshape(-1).astype(jnp.int32)
    oc = jnp.concatenate([oc, jnp.zeros((n_pad - n,), jnp.int32)])

    e = _sc_gather(oc, table, n_pad, n_chunks)

    w_e = W[:ne]
    w_f = W[ne:ne + nf]
    w_d = W[ne + nf:ne + nf + nd]
    w_c = W[ne + nf + nd:]

    grid = (-(-n // _BLK),)
    row_block = lambda width: pl.BlockSpec((_BLK, width), lambda i: (i, 0))
    full = lambda a: pl.BlockSpec(a.shape, lambda i: (0, 0))

    out = pl.pallas_call(
        _tc_body,
        grid=grid,
        in_specs=[
            row_block(ne),
            row_block(nf),
            row_block(nd),
            row_block(ncf),
            full(w_e), full(w_f), full(w_d), full(w_c),
            pl.BlockSpec((1, out_ch), lambda i: (0, 0)),
            pl.BlockSpec((1, out_ch), lambda i: (0, 0)),
            pl.BlockSpec((1, out_ch), lambda i: (0, 0)),
        ],
        out_specs=pl.BlockSpec((_BLK, out_ch), lambda i: (i, 0)),
        out_shape=jax.ShapeDtypeStruct((n, out_ch), jnp.float32),
    )(e, features, dim_features, configs, w_e, w_f, w_d, w_c,
      b.reshape(1, -1), gamma.reshape(1, -1), beta.reshape(1, -1))
    return out
